# baseline (speedup 1.0000x reference)
import functools

import jax
import jax.numpy as jnp
from jax import lax
from jax.experimental import pallas as pl
from jax.experimental.pallas import tpu as pltpu
from jax.experimental.pallas import tpu_sc as plsc

B = 4
L = 256
R = 256
D_SRC = 128
E_POS = 64
E_CAT = 64
E_SENSE = 64
E_DM = E_POS + E_CAT + E_SENSE
REL = 256
INP = E_DM + D_SRC


def _sc_gather_body(f0, f1, f2, index_hbm, src_flat, pos_lut, cat_lut,
                    sense_lut, pos_out, cat_out, sense_out, hsrc_out,
                    idx_v, rows128, gidx_v, sem):
    nw = 32
    chunk = (B * L) // nw
    wid = lax.axis_index("s") * 2 + lax.axis_index("c")
    base = wid * chunk

    def lut_gather(feat_col, lut, out):
        pltpu.sync_copy(feat_col.at[pl.ds(base, chunk)], idx_v)
        pltpu.async_copy(lut.at[idx_v], rows128, sem).wait()
        pltpu.sync_copy(rows128, out.at[pl.ds(base, chunk)])

    lut_gather(f0, pos_lut, pos_out)
    lut_gather(f1, cat_lut, cat_out)
    lut_gather(f2, sense_lut, sense_out)

    pltpu.sync_copy(index_hbm.at[pl.ds(base, chunk)], idx_v)
    for j in range(chunk // 16):
        row_ids = (lax.iota(jnp.int32, 16) + (base + j * 16)) * R
        gidx_v[pl.ds(j * 16, 16)] = row_ids + idx_v[pl.ds(j * 16, 16)]
    pltpu.async_copy(src_flat.at[gidx_v], rows128, sem).wait()
    pltpu.sync_copy(rows128, hsrc_out.at[pl.ds(base, chunk)])


def _sc_gather(f0, f1, f2, index, src_flat, pos_lut, cat_lut, sense_lut):
    nw = 32
    chunk = (B * L) // nw
    mesh = plsc.VectorSubcoreMesh(core_axis_name="c", subcore_axis_name="s")
    fn = pl.kernel(
        _sc_gather_body, mesh=mesh,
        out_type=[
            jax.ShapeDtypeStruct((B * L, 128), jnp.float32),
            jax.ShapeDtypeStruct((B * L, 128), jnp.float32),
            jax.ShapeDtypeStruct((B * L, 128), jnp.float32),
            jax.ShapeDtypeStruct((B * L, D_SRC), jnp.float32),
        ],
        scratch_types=[
            pltpu.VMEM((chunk,), jnp.int32),
            pltpu.VMEM((chunk, D_SRC), jnp.float32),
            pltpu.VMEM((chunk,), jnp.int32),
            pltpu.SemaphoreType.DMA,
        ],
    )
    return fn(f0, f1, f2, index, src_flat, pos_lut, cat_lut, sense_lut)


def _head_body(dm_ref, hsrc_ref, whdm_ref, whsrc_ref, wddm_ref, hb_ref,
               db_ref, hout_ref, dmdep_ref):
    dm = dm_ref[...]
    hout_ref[...] = (
        jnp.dot(dm, whdm_ref[...], preferred_element_type=jnp.float32)
        + jnp.dot(hsrc_ref[...], whsrc_ref[...],
                  preferred_element_type=jnp.float32)
        + hb_ref[...]
    )
    dmdep_ref[...] = (
        jnp.dot(dm, wddm_ref[...], preferred_element_type=jnp.float32)
        + db_ref[...]
    )


TA = 64


def _dep_body(idx_ref, enc_ref, dmdep_ref, wsrc_ref, out_ref, gt_ref):
    t = pl.program_id(1)

    @pl.when(t == 0)
    def _():
        idx = idx_ref[0, 0, :]
        rows = lax.broadcasted_iota(jnp.int32, (R, L), 0)
        gt_ref[...] = (rows == idx[None, :]).astype(jnp.bfloat16)

    gt = gt_ref[...]
    w = wsrc_ref[...]
    dm = dmdep_ref[0]
    for i in range(TA):
        enc = enc_ref[0, i].astype(jnp.bfloat16)
        gathered = lax.dot_general(
            gt, enc, (((0,), (0,)), ((), ())),
            preferred_element_type=jnp.float32)
        dep = jnp.dot(gathered.astype(jnp.bfloat16), w,
                      preferred_element_type=jnp.float32)
        out_ref[0, i] = dep + dm


def kernel(feats, index, src_enc, pos_lut, cat_lut, sense_lut, head_W,
           head_b, dep_W, dep_b):
    f0, f1, f2 = (feats[:, i] for i in range(3))
    src_flat = src_enc.reshape(B * L * R, D_SRC)
    pad = ((0, 0), (0, 128 - E_POS))
    pos_p = jnp.pad(pos_lut, pad)
    cat_p = jnp.pad(cat_lut, pad)
    sense_p = jnp.pad(sense_lut, pad)

    pos_e, cat_e, sense_e, head_src = _sc_gather(
        f0, f1, f2, index, src_flat, pos_p, cat_p, sense_p)
    dm_emb = jnp.concatenate(
        [pos_e[:, :E_POS], cat_e[:, :E_CAT], sense_e[:, :E_SENSE]], axis=1)

    head_Wt = head_W.T
    dep_Wt = dep_W.T

    head_out, dmdep = pl.pallas_call(
        _head_body,
        out_shape=[
            jax.ShapeDtypeStruct((B * L, REL), jnp.float32),
            jax.ShapeDtypeStruct((B * L, REL), jnp.float32),
        ],
    )(dm_emb, head_src, head_Wt[:E_DM], head_Wt[E_DM:], dep_Wt[:E_DM],
      head_b.reshape(1, REL), dep_b.reshape(1, REL))

    idx3 = index.reshape(B, 1, L)
    enc4 = src_enc.reshape(B, L, R, D_SRC)
    dmdep3 = dmdep.reshape(B, L, REL)

    dep_out = pl.pallas_call(
        _dep_body,
        grid=(B, L // TA),
        in_specs=[
            pl.BlockSpec((1, 1, L), lambda b, t: (b, 0, 0)),
            pl.BlockSpec((1, TA, R, D_SRC), lambda b, t: (b, t, 0, 0)),
            pl.BlockSpec((1, L, REL), lambda b, t: (b, 0, 0)),
            pl.BlockSpec((D_SRC, REL), lambda b, t: (0, 0)),
        ],
        out_specs=pl.BlockSpec((1, TA, L, REL), lambda b, t: (b, t, 0, 0)),
        out_shape=jax.ShapeDtypeStruct((B, L, L, REL), jnp.float32),
        scratch_shapes=[pltpu.VMEM((R, L), jnp.bfloat16)],
    )(idx3, enc4, dmdep3, dep_Wt[E_DM:].astype(jnp.bfloat16))

    return (dm_emb, head_out, dep_out.reshape(B * L, L, REL))


# speedup vs baseline: 1.0458x; 1.0458x over previous
import jax
import jax.numpy as jnp
from jax import lax
from jax.experimental import pallas as pl
from jax.experimental.pallas import tpu as pltpu
from jax.experimental.pallas import tpu_sc as plsc

B = 4
L = 256
R = 256
D_SRC = 128
E_POS = 64
E_CAT = 64
E_SENSE = 64
E_DM = E_POS + E_CAT + E_SENSE
REL = 256
INP = E_DM + D_SRC
VPAD = 64


def _sc_head_body(index_hbm, src_flat, hsrc_out, idx_v, rows_v, gidx_v, sem):
    nw = 32
    chunk = (B * L) // nw
    wid = lax.axis_index("s") * 2 + lax.axis_index("c")
    base = wid * chunk

    pltpu.sync_copy(index_hbm.at[pl.ds(base, chunk)], idx_v)
    for j in range(chunk // 16):
        row_ids = (lax.iota(jnp.int32, 16) + (base + j * 16)) * R
        gidx_v[pl.ds(j * 16, 16)] = row_ids + idx_v[pl.ds(j * 16, 16)]
    pltpu.async_copy(src_flat.at[gidx_v], rows_v, sem).wait()
    pltpu.sync_copy(rows_v, hsrc_out.at[pl.ds(base, chunk)])


def _sc_head_gather(index, src_flat):
    chunk = (B * L) // 32
    mesh = plsc.VectorSubcoreMesh(core_axis_name="c", subcore_axis_name="s")
    fn = pl.kernel(
        _sc_head_body, mesh=mesh,
        out_type=jax.ShapeDtypeStruct((B * L, D_SRC), jnp.float32),
        scratch_types=[
            pltpu.VMEM((chunk,), jnp.int32),
            pltpu.VMEM((chunk, D_SRC), jnp.float32),
            pltpu.VMEM((chunk,), jnp.int32),
            pltpu.SemaphoreType.DMA,
        ],
    )
    return fn(index, src_flat)


def _dm_body(feats_ref, pos_ref, cat_ref, sense_ref, wddm_ref, db_ref,
             dm_ref, dmdep_ref):
    cols = lax.broadcasted_iota(jnp.int32, (B * L, VPAD), 1)

    def lut(col, table_ref):
        oh = (cols == feats_ref[:, col:col + 1]).astype(jnp.float32)
        return jnp.dot(oh, table_ref[...], preferred_element_type=jnp.float32)

    dm = jnp.concatenate(
        [lut(0, pos_ref), lut(1, cat_ref), lut(2, sense_ref)], axis=1)
    dm_ref[...] = dm
    dmdep_ref[...] = (
        jnp.dot(dm.astype(jnp.bfloat16), wddm_ref[...],
                preferred_element_type=jnp.float32)
        + db_ref[...]
    )


def _head_body(dm_ref, hsrc_ref, whdm_ref, whsrc_ref, hb_ref, hout_ref):
    hout_ref[...] = (
        jnp.dot(dm_ref[...], whdm_ref[...], preferred_element_type=jnp.float32)
        + jnp.dot(hsrc_ref[...], whsrc_ref[...],
                  preferred_element_type=jnp.float32)
        + hb_ref[...]
    )


TA = 64


def _dep_body(idx_ref, enc_ref, dmdep_ref, wsrc_ref, out_ref, gt_ref):
    t = pl.program_id(1)

    @pl.when(t == 0)
    def _():
        idx = idx_ref[0, 0, :]
        rows = lax.broadcasted_iota(jnp.int32, (R, L), 0)
        gt_ref[...] = (rows == idx[None, :]).astype(jnp.bfloat16)

    gt = gt_ref[...]
    w = wsrc_ref[...]
    dm = dmdep_ref[0]
    for i in range(TA):
        enc = enc_ref[0, i].astype(jnp.bfloat16)
        gathered = lax.dot_general(
            gt, enc, (((0,), (0,)), ((), ())),
            preferred_element_type=jnp.float32)
        dep = jnp.dot(gathered.astype(jnp.bfloat16), w,
                      preferred_element_type=jnp.float32)
        out_ref[0, i] = dep + dm


def kernel(feats, index, src_enc, pos_lut, cat_lut, sense_lut, head_W,
           head_b, dep_W, dep_b):
    src_flat = src_enc.reshape(B * L * R, D_SRC)
    vp = ((0, VPAD - 50), (0, 0))

    head_src = _sc_head_gather(index, src_flat)

    head_Wt = head_W.T
    dep_Wt = dep_W.T

    dm_emb, dmdep = pl.pallas_call(
        _dm_body,
        out_shape=[
            jax.ShapeDtypeStruct((B * L, E_DM), jnp.float32),
            jax.ShapeDtypeStruct((B * L, REL), jnp.float32),
        ],
    )(feats, jnp.pad(pos_lut, vp), jnp.pad(cat_lut, vp),
      jnp.pad(sense_lut, vp), dep_Wt[:E_DM].astype(jnp.bfloat16),
      dep_b.reshape(1, REL))

    idx3 = index.reshape(B, 1, L)
    enc4 = src_enc.reshape(B, L, R, D_SRC)
    dmdep3 = dmdep.reshape(B, L, REL)

    dep_out = pl.pallas_call(
        _dep_body,
        grid=(B, L // TA),
        in_specs=[
            pl.BlockSpec((1, 1, L), lambda b, t: (b, 0, 0)),
            pl.BlockSpec((1, TA, R, D_SRC), lambda b, t: (b, t, 0, 0)),
            pl.BlockSpec((1, L, REL), lambda b, t: (b, 0, 0)),
            pl.BlockSpec((D_SRC, REL), lambda b, t: (0, 0)),
        ],
        out_specs=pl.BlockSpec((1, TA, L, REL), lambda b, t: (b, t, 0, 0)),
        out_shape=jax.ShapeDtypeStruct((B, L, L, REL), jnp.float32),
        scratch_shapes=[pltpu.VMEM((R, L), jnp.bfloat16)],
    )(idx3, enc4, dmdep3, dep_Wt[E_DM:].astype(jnp.bfloat16))

    head_out = pl.pallas_call(
        _head_body,
        out_shape=jax.ShapeDtypeStruct((B * L, REL), jnp.float32),
    )(dm_emb, head_src, head_Wt[:E_DM], head_Wt[E_DM:],
      head_b.reshape(1, REL))

    return (dm_emb, head_out, dep_out.reshape(B * L, L, REL))
